# Initial kernel scaffold; baseline (speedup 1.0000x reference)
import jax
import jax.numpy as jnp
from jax.experimental import pallas as pl


def kernel(x, edge_index, W1l, b1l, W1r, W2l, b2l, W2r):
    raise NotImplementedError("write your pallas kernel here")


# speedup vs baseline: 4.6530x; 4.6530x over previous
import functools

import jax
import jax.numpy as jnp
from jax import lax
from jax.experimental import pallas as pl
from jax.experimental.pallas import tpu as pltpu
from jax.experimental.pallas import tpu_sc as plsc

N = 10000
E = 320000
D = 128

NC = 2
NS = 16
NW = NC * NS

CHUNK = 128
CPT = 80
EP = NW * CPT * CHUNK
NP = 10112
NBLK = NP // CHUNK
KMAX = (NBLK + NS - 1) // NS

CCH = 128
CCPT = EP // (NW * CCH)
SCH = 16
SCPT = EP // (NW * SCH)
CW = 16
CBLK = NP // CCH
CKMAX = (CBLK + NS - 1) // NS

_MESH = plsc.VectorSubcoreMesh(core_axis_name="c", subcore_axis_name="s")


def _sc_agg_body(table_hbm, src_hbm, dst_hbm, zeros_hbm,
                 parts_hbm, sidx, didx, rbuf, acc, gsem):
    cid = lax.axis_index("c")
    sid = lax.axis_index("s")
    wid = sid * NC + cid
    pltpu.sync_copy(zeros_hbm, rbuf)

    def zero_body(k, c):
        b = sid + k * NS

        @pl.when(b < NBLK)
        def _():
            pltpu.sync_copy(rbuf, acc.at[pl.ds(b * CHUNK, CHUNK)])

        return c

    lax.fori_loop(0, KMAX, zero_body, 0)
    plsc.subcore_barrier()

    def chunk_body(j, c):
        pltpu.sync_copy(src_hbm.at[wid, j], sidx)
        pltpu.sync_copy(dst_hbm.at[wid, j], didx)
        pltpu.async_copy(table_hbm.at[sidx], rbuf, gsem).wait()
        pltpu.sync_copy(rbuf, acc.at[didx], add=True)
        return c

    lax.fori_loop(0, CPT, chunk_body, 0)
    plsc.subcore_barrier()

    def out_body(k, c):
        b = sid + k * NS

        @pl.when(b < NBLK)
        def _():
            r = b * CHUNK
            pltpu.sync_copy(acc.at[pl.ds(r, CHUNK)], rbuf)
            pltpu.sync_copy(rbuf, parts_hbm.at[cid, pl.ds(r, CHUNK)])

        return c

    lax.fori_loop(0, KMAX, out_body, 0)


_sc_agg = pl.kernel(
    _sc_agg_body,
    out_type=jax.ShapeDtypeStruct((NC, NP, D), jnp.float32),
    mesh=_MESH,
    scratch_types=[
        pltpu.VMEM((CHUNK,), jnp.int32),
        pltpu.VMEM((CHUNK,), jnp.int32),
        pltpu.VMEM((CHUNK, D), jnp.float32),
        pltpu.VMEM_SHARED((NP, D), jnp.float32),
        pltpu.SemaphoreType.DMA,
    ],
)


def _sc_count_body(dst_hbm, ones_hbm, zeros_hbm,
                   cnts_hbm, didx2, onev, cbuf, cacc):
    cid = lax.axis_index("c")
    sid = lax.axis_index("s")
    wid = sid * NC + cid
    pltpu.sync_copy(ones_hbm, onev)
    pltpu.sync_copy(zeros_hbm, cbuf)
    pltpu.sync_copy(dst_hbm.at[wid], didx2)

    def zero_body(k, c):
        b = sid + k * NS

        @pl.when(b < CBLK)
        def _():
            pltpu.sync_copy(cbuf, cacc.at[pl.ds(b * CCH, CCH)])

        return c

    lax.fori_loop(0, CKMAX, zero_body, 0)
    plsc.subcore_barrier()

    idxv = didx2[0, pl.ds(0, SCH)]
    pltpu.sync_copy(onev, cacc.at[idxv], add=True)
    plsc.subcore_barrier()

    def out_body(k, c):
        b = sid + k * NS

        @pl.when(b < CBLK)
        def _():
            r = b * CCH
            pltpu.sync_copy(cacc.at[pl.ds(r, CCH)], cbuf)
            pltpu.sync_copy(cbuf, cnts_hbm.at[cid, pl.ds(r, CCH)])

        return c

    lax.fori_loop(0, CKMAX, out_body, 0)


_sc_count = pl.kernel(
    _sc_count_body,
    out_type=jax.ShapeDtypeStruct((NC, NP, CW), jnp.float32),
    mesh=_MESH,
    scratch_types=[
        pltpu.VMEM((CCPT, CCH), jnp.int32),
        pltpu.VMEM((SCH, CW), jnp.float32),
        pltpu.VMEM((CCH, CW), jnp.float32),
        pltpu.VMEM_SHARED((NP, CW), jnp.float32),
    ],
)

def _sc_diag_body(dst_hbm, ones_hbm, oidx_hbm, oone_hbm, didx, onev):
    cid = lax.axis_index("c")
    sid = lax.axis_index("s")
    wid = sid * NC + cid
    pltpu.sync_copy(ones_hbm, onev)
    pltpu.sync_copy(onev, oone_hbm.at[wid])

    def chunk_body(j, c):
        pltpu.sync_copy(dst_hbm.at[wid, j], didx)
        pltpu.sync_copy(didx, oidx_hbm.at[wid, j])
        return c

    lax.fori_loop(0, CCPT, chunk_body, 0)


_sc_diag = pl.kernel(
    _sc_diag_body,
    out_type=(jax.ShapeDtypeStruct((NW, CCPT, CCH), jnp.int32),
              jax.ShapeDtypeStruct((NW, CCH, CW), jnp.float32)),
    mesh=_MESH,
    scratch_types=[
        pltpu.VMEM((CCH,), jnp.int32),
        pltpu.VMEM((CCH, CW), jnp.float32),
    ],
)

BN = 400


def _dot_t(a, w):
    return lax.dot_general(a, w, (((1,), (1,)), ((), ())),
                           preferred_element_type=jnp.float32)


def _dense1_body(p_ref, c_ref, x_ref, wl_ref, bl_ref, wr_ref, h_ref, ic_ref):
    cnt = c_ref[0, :, 0] + c_ref[1, :, 0]
    invc = 1.0 / jnp.maximum(cnt, 1.0)
    m = (p_ref[0] + p_ref[1]) * invc[:, None]
    h = _dot_t(m, wl_ref[...]) + bl_ref[...][None, :] + _dot_t(x_ref[...], wr_ref[...])
    h_ref[...] = jnp.maximum(h, 0.0)
    ic_ref[...] = invc[:, None]


def _dense2_body(p_ref, ic_ref, h_ref, wl_ref, bl_ref, wr_ref, o_ref):
    m = (p_ref[0] + p_ref[1]) * ic_ref[...]
    o_ref[...] = _dot_t(m, wl_ref[...]) + bl_ref[...][None, :] + _dot_t(h_ref[...], wr_ref[...])


def _dense1(parts, cnts, x, Wl, bl, Wr):
    return pl.pallas_call(
        _dense1_body,
        grid=(N // BN,),
        in_specs=[
            pl.BlockSpec((NC, BN, D), lambda i: (0, i, 0)),
            pl.BlockSpec((NC, BN, D), lambda i: (0, i, 0)),
            pl.BlockSpec((BN, D), lambda i: (i, 0)),
            pl.BlockSpec((D, D), lambda i: (0, 0)),
            pl.BlockSpec((D,), lambda i: (0,)),
            pl.BlockSpec((D, D), lambda i: (0, 0)),
        ],
        out_specs=[
            pl.BlockSpec((BN, D), lambda i: (i, 0)),
            pl.BlockSpec((BN, 1), lambda i: (i, 0)),
        ],
        out_shape=[
            jax.ShapeDtypeStruct((N, D), jnp.float32),
            jax.ShapeDtypeStruct((N, 1), jnp.float32),
        ],
    )(parts, cnts, x, Wl, bl, Wr)


def _dense2(parts, invc, h, Wl, bl, Wr):
    return pl.pallas_call(
        _dense2_body,
        grid=(N // BN,),
        in_specs=[
            pl.BlockSpec((NC, BN, D), lambda i: (0, i, 0)),
            pl.BlockSpec((BN, 1), lambda i: (i, 0)),
            pl.BlockSpec((BN, D), lambda i: (i, 0)),
            pl.BlockSpec((D, D), lambda i: (0, 0)),
            pl.BlockSpec((D,), lambda i: (0,)),
            pl.BlockSpec((D, D), lambda i: (0, 0)),
        ],
        out_specs=pl.BlockSpec((BN, D), lambda i: (i, 0)),
        out_shape=jax.ShapeDtypeStruct((N, D), jnp.float32),
    )(parts, invc, h, Wl, bl, Wr)


def kernel(x, edge_index, W1l, b1l, W1r, W2l, b2l, W2r):
    src = edge_index[0].astype(jnp.int32)
    dst = edge_index[1].astype(jnp.int32)
    pad = EP - E
    pad_ar = jnp.arange(pad, dtype=jnp.int32)
    pad_src = (pad_ar * 97) % N
    pad_dst = N + pad_ar % (NP - N)
    src = jnp.concatenate([src, pad_src]).reshape(NW, CPT, CHUNK)
    dst_flat = jnp.concatenate([dst, pad_dst])
    dst = dst_flat.reshape(NW, CPT, CHUNK)
    dst_c = dst_flat.reshape(NW, CCPT, CCH)
    zeros_d = jnp.zeros((CHUNK, D), jnp.float32)
    zeros_c = jnp.zeros((CCH, CW), jnp.float32)
    ones_c = jnp.ones((SCH, CW), jnp.float32)

    ones_table = jnp.ones((N, D), jnp.float32)
    cnts = _sc_agg(ones_table, src, dst, zeros_d)
    parts1 = _sc_agg(x, src, dst, zeros_d)
    h, invc = _dense1(parts1, cnts, x, W1l, b1l, W1r)
    parts2 = _sc_agg(h, src, dst, zeros_d)
    out = _dense2(parts2, invc, h, W2l, b2l, W2r)
    return out


# speedup vs baseline: 5.6482x; 1.2139x over previous
import functools

import jax
import jax.numpy as jnp
from jax import lax
from jax.experimental import pallas as pl
from jax.experimental.pallas import tpu as pltpu
from jax.experimental.pallas import tpu_sc as plsc

N = 10000
E = 320000
D = 128

NC = 2
NS = 16
NW = NC * NS

CHUNK = 128
CPT = 80
EP = NW * CPT * CHUNK
NP = 10112
NBLK = NP // CHUNK
KMAX = (NBLK + NS - 1) // NS

CCH = 128
CCPT = EP // (NW * CCH)
SCH = 16
SCPT = EP // (NW * SCH)
CW = 16
CBLK = NP // CCH
CKMAX = (CBLK + NS - 1) // NS

_MESH = plsc.VectorSubcoreMesh(core_axis_name="c", subcore_axis_name="s")


def _sc_agg_body(table_hbm, src_hbm, dst_hbm, zeros_hbm,
                 parts_hbm, sidx, didx, rbuf, acc, gsem):
    cid = lax.axis_index("c")
    sid = lax.axis_index("s")
    wid = sid * NC + cid
    pltpu.sync_copy(zeros_hbm, rbuf)

    def zero_body(k, c):
        b = sid + k * NS

        @pl.when(b < NBLK)
        def _():
            pltpu.sync_copy(rbuf, acc.at[pl.ds(b * CHUNK, CHUNK)])

        return c

    lax.fori_loop(0, KMAX, zero_body, 0)
    plsc.subcore_barrier()

    def chunk_body(j, c):
        pltpu.sync_copy(src_hbm.at[wid, j], sidx)
        pltpu.sync_copy(dst_hbm.at[wid, j], didx)
        pltpu.async_copy(table_hbm.at[sidx], rbuf, gsem).wait()
        pltpu.sync_copy(rbuf, acc.at[didx], add=True)
        return c

    lax.fori_loop(0, CPT, chunk_body, 0)
    plsc.subcore_barrier()

    def out_body(k, c):
        b = sid + k * NS

        @pl.when(b < NBLK)
        def _():
            r = b * CHUNK
            pltpu.sync_copy(acc.at[pl.ds(r, CHUNK)], rbuf)
            pltpu.sync_copy(rbuf, parts_hbm.at[cid, pl.ds(r, CHUNK)])

        return c

    lax.fori_loop(0, KMAX, out_body, 0)


_sc_agg = pl.kernel(
    _sc_agg_body,
    out_type=jax.ShapeDtypeStruct((NC, NP, D), jnp.float32),
    mesh=_MESH,
    scratch_types=[
        pltpu.VMEM((CHUNK,), jnp.int32),
        pltpu.VMEM((CHUNK,), jnp.int32),
        pltpu.VMEM((CHUNK, D), jnp.float32),
        pltpu.VMEM_SHARED((NP, D), jnp.float32),
        pltpu.SemaphoreType.DMA,
    ],
)


def _sc_ones_body(dst_hbm, ones_hbm, zeros_hbm, cnts_hbm, didx, rbuf, acc):
    cid = lax.axis_index("c")
    sid = lax.axis_index("s")
    wid = sid * NC + cid
    pltpu.sync_copy(zeros_hbm, rbuf)

    def zero_body(k, c):
        b = sid + k * NS

        @pl.when(b < NBLK)
        def _():
            pltpu.sync_copy(rbuf, acc.at[pl.ds(b * CHUNK, CHUNK)])

        return c

    lax.fori_loop(0, KMAX, zero_body, 0)
    pltpu.sync_copy(ones_hbm, rbuf)
    plsc.subcore_barrier()

    def chunk_body(j, c):
        pltpu.sync_copy(dst_hbm.at[wid, j], didx)
        pltpu.sync_copy(rbuf, acc.at[didx], add=True)
        return c

    lax.fori_loop(0, CPT, chunk_body, 0)
    plsc.subcore_barrier()

    def out_body(k, c):
        b = sid + k * NS

        @pl.when(b < NBLK)
        def _():
            r = b * CHUNK
            pltpu.sync_copy(acc.at[pl.ds(r, CHUNK)], rbuf)
            pltpu.sync_copy(rbuf, cnts_hbm.at[cid, pl.ds(r, CHUNK)])

        return c

    lax.fori_loop(0, KMAX, out_body, 0)


_sc_ones = pl.kernel(
    _sc_ones_body,
    out_type=jax.ShapeDtypeStruct((NC, NP, D), jnp.float32),
    mesh=_MESH,
    scratch_types=[
        pltpu.VMEM((CHUNK,), jnp.int32),
        pltpu.VMEM((CHUNK, D), jnp.float32),
        pltpu.VMEM_SHARED((NP, D), jnp.float32),
    ],
)


BN = 400


def _dot_t(a, w):
    return lax.dot_general(a, w, (((1,), (1,)), ((), ())),
                           preferred_element_type=jnp.float32)


def _dense1_body(p_ref, c_ref, x_ref, wl_ref, bl_ref, wr_ref, h_ref, ic_ref):
    cnt = c_ref[0, :, 0] + c_ref[1, :, 0]
    invc = 1.0 / jnp.maximum(cnt, 1.0)
    m = (p_ref[0] + p_ref[1]) * invc[:, None]
    h = _dot_t(m, wl_ref[...]) + bl_ref[...][None, :] + _dot_t(x_ref[...], wr_ref[...])
    h_ref[...] = jnp.maximum(h, 0.0)
    ic_ref[...] = invc[:, None]


def _dense2_body(p_ref, ic_ref, h_ref, wl_ref, bl_ref, wr_ref, o_ref):
    m = (p_ref[0] + p_ref[1]) * ic_ref[...]
    o_ref[...] = _dot_t(m, wl_ref[...]) + bl_ref[...][None, :] + _dot_t(h_ref[...], wr_ref[...])


def _dense1(parts, cnts, x, Wl, bl, Wr):
    return pl.pallas_call(
        _dense1_body,
        grid=(N // BN,),
        in_specs=[
            pl.BlockSpec((NC, BN, D), lambda i: (0, i, 0)),
            pl.BlockSpec((NC, BN, D), lambda i: (0, i, 0)),
            pl.BlockSpec((BN, D), lambda i: (i, 0)),
            pl.BlockSpec((D, D), lambda i: (0, 0)),
            pl.BlockSpec((D,), lambda i: (0,)),
            pl.BlockSpec((D, D), lambda i: (0, 0)),
        ],
        out_specs=[
            pl.BlockSpec((BN, D), lambda i: (i, 0)),
            pl.BlockSpec((BN, 1), lambda i: (i, 0)),
        ],
        out_shape=[
            jax.ShapeDtypeStruct((N, D), jnp.float32),
            jax.ShapeDtypeStruct((N, 1), jnp.float32),
        ],
    )(parts, cnts, x, Wl, bl, Wr)


def _dense2(parts, invc, h, Wl, bl, Wr):
    return pl.pallas_call(
        _dense2_body,
        grid=(N // BN,),
        in_specs=[
            pl.BlockSpec((NC, BN, D), lambda i: (0, i, 0)),
            pl.BlockSpec((BN, 1), lambda i: (i, 0)),
            pl.BlockSpec((BN, D), lambda i: (i, 0)),
            pl.BlockSpec((D, D), lambda i: (0, 0)),
            pl.BlockSpec((D,), lambda i: (0,)),
            pl.BlockSpec((D, D), lambda i: (0, 0)),
        ],
        out_specs=pl.BlockSpec((BN, D), lambda i: (i, 0)),
        out_shape=jax.ShapeDtypeStruct((N, D), jnp.float32),
    )(parts, invc, h, Wl, bl, Wr)


def kernel(x, edge_index, W1l, b1l, W1r, W2l, b2l, W2r):
    src = edge_index[0].astype(jnp.int32)
    dst = edge_index[1].astype(jnp.int32)
    pad = EP - E
    pad_ar = jnp.arange(pad, dtype=jnp.int32)
    pad_src = (pad_ar * 97) % N
    pad_dst = N + pad_ar % (NP - N)
    src = jnp.concatenate([src, pad_src]).reshape(NW, CPT, CHUNK)
    dst = jnp.concatenate([dst, pad_dst]).reshape(NW, CPT, CHUNK)
    zeros_d = jnp.zeros((CHUNK, D), jnp.float32)

    ones_d = jnp.ones((CHUNK, D), jnp.float32)
    cnts = _sc_ones(dst, ones_d, zeros_d)
    parts1 = _sc_agg(x, src, dst, zeros_d)
    h, invc = _dense1(parts1, cnts, x, W1l, b1l, W1r)
    parts2 = _sc_agg(h, src, dst, zeros_d)
    out = _dense2(parts2, invc, h, W2l, b2l, W2r)
    return out


# speedup vs baseline: 5.9138x; 1.0470x over previous
import functools

import jax
import jax.numpy as jnp
from jax import lax
from jax.experimental import pallas as pl
from jax.experimental.pallas import tpu as pltpu
from jax.experimental.pallas import tpu_sc as plsc

N = 10000
E = 320000
D = 128

NC = 2
NS = 16
NW = NC * NS

CHUNK = 128
CPT = 80
EP = NW * CPT * CHUNK
NP = 10112
NBLK = NP // CHUNK
KMAX = (NBLK + NS - 1) // NS

CCH = 128
CCPT = EP // (NW * CCH)
SCH = 16
SCPT = EP // (NW * SCH)
CW = 16
CBLK = NP // CCH
CKMAX = (CBLK + NS - 1) // NS

_MESH = plsc.VectorSubcoreMesh(core_axis_name="c", subcore_axis_name="s")


CH2 = 64
CPT2 = EP // (NW * CH2)
NB2 = NP // CH2
KMAX2 = (NB2 + NS - 1) // NS


def _sc_agg_body(table_hbm, src_hbm, dst_hbm, zeros_hbm, parts_hbm,
                 sidx0, didx0, rbuf0, gsem0, sidx1, didx1, rbuf1, gsem1, acc):
    cid = lax.axis_index("c")
    sid = lax.axis_index("s")
    wid = sid * NC + cid
    pltpu.sync_copy(zeros_hbm, rbuf0)
    pltpu.sync_copy(zeros_hbm, rbuf1)

    def zero_body(k, c):
        b = sid + k * NS

        @pl.when(b < NB2)
        def _():
            pltpu.sync_copy(rbuf0, acc.at[pl.ds(b * CH2, CH2)])

        return c

    lax.fori_loop(0, KMAX2, zero_body, 0)
    plsc.subcore_barrier()

    pltpu.sync_copy(src_hbm.at[wid, 0], sidx0)
    pltpu.sync_copy(dst_hbm.at[wid, 0], didx0)
    pltpu.async_copy(table_hbm.at[sidx0], rbuf0, gsem0)

    def pair_body(g, c):
        j1 = 2 * g + 1
        pltpu.sync_copy(src_hbm.at[wid, j1], sidx1)
        pltpu.sync_copy(dst_hbm.at[wid, j1], didx1)
        pltpu.async_copy(table_hbm.at[sidx1], rbuf1, gsem1)
        pltpu.make_async_copy(table_hbm.at[sidx0], rbuf0, gsem0).wait()
        pltpu.sync_copy(rbuf0, acc.at[didx0], add=True)

        @pl.when(j1 + 1 < CPT2)
        def _():
            pltpu.sync_copy(src_hbm.at[wid, j1 + 1], sidx0)
            pltpu.sync_copy(dst_hbm.at[wid, j1 + 1], didx0)
            pltpu.async_copy(table_hbm.at[sidx0], rbuf0, gsem0)

        pltpu.make_async_copy(table_hbm.at[sidx1], rbuf1, gsem1).wait()
        pltpu.sync_copy(rbuf1, acc.at[didx1], add=True)
        return c

    lax.fori_loop(0, CPT2 // 2, pair_body, 0)
    plsc.subcore_barrier()

    def out_body(k, c):
        b = sid + k * NS

        @pl.when(b < NB2)
        def _():
            r = b * CH2
            pltpu.sync_copy(acc.at[pl.ds(r, CH2)], rbuf0)
            pltpu.sync_copy(rbuf0, parts_hbm.at[cid, pl.ds(r, CH2)])

        return c

    lax.fori_loop(0, KMAX2, out_body, 0)


_sc_agg = pl.kernel(
    _sc_agg_body,
    out_type=jax.ShapeDtypeStruct((NC, NP, D), jnp.float32),
    mesh=_MESH,
    scratch_types=[
        pltpu.VMEM((CH2,), jnp.int32),
        pltpu.VMEM((CH2,), jnp.int32),
        pltpu.VMEM((CH2, D), jnp.float32),
        pltpu.SemaphoreType.DMA,
        pltpu.VMEM((CH2,), jnp.int32),
        pltpu.VMEM((CH2,), jnp.int32),
        pltpu.VMEM((CH2, D), jnp.float32),
        pltpu.SemaphoreType.DMA,
        pltpu.VMEM_SHARED((NP, D), jnp.float32),
    ],
)


def _sc_ones_body(dst_hbm, ones_hbm, zeros_hbm, cnts_hbm, didx, rbuf, acc):
    cid = lax.axis_index("c")
    sid = lax.axis_index("s")
    wid = sid * NC + cid
    pltpu.sync_copy(zeros_hbm, rbuf)

    def zero_body(k, c):
        b = sid + k * NS

        @pl.when(b < NBLK)
        def _():
            pltpu.sync_copy(rbuf, acc.at[pl.ds(b * CHUNK, CHUNK)])

        return c

    lax.fori_loop(0, KMAX, zero_body, 0)
    pltpu.sync_copy(ones_hbm, rbuf)
    plsc.subcore_barrier()

    def chunk_body(j, c):
        pltpu.sync_copy(dst_hbm.at[wid, j], didx)
        pltpu.sync_copy(rbuf, acc.at[didx], add=True)
        return c

    lax.fori_loop(0, CPT, chunk_body, 0)
    plsc.subcore_barrier()

    def out_body(k, c):
        b = sid + k * NS

        @pl.when(b < NBLK)
        def _():
            r = b * CHUNK
            pltpu.sync_copy(acc.at[pl.ds(r, CHUNK)], rbuf)
            pltpu.sync_copy(rbuf, cnts_hbm.at[cid, pl.ds(r, CHUNK)])

        return c

    lax.fori_loop(0, KMAX, out_body, 0)


_sc_ones = pl.kernel(
    _sc_ones_body,
    out_type=jax.ShapeDtypeStruct((NC, NP, D), jnp.float32),
    mesh=_MESH,
    scratch_types=[
        pltpu.VMEM((CHUNK,), jnp.int32),
        pltpu.VMEM((CHUNK, D), jnp.float32),
        pltpu.VMEM_SHARED((NP, D), jnp.float32),
    ],
)


BN = 400


def _dot_t(a, w):
    return lax.dot_general(a, w, (((1,), (1,)), ((), ())),
                           preferred_element_type=jnp.float32)


def _dense1_body(p_ref, c_ref, x_ref, wl_ref, bl_ref, wr_ref, h_ref, ic_ref):
    cnt = c_ref[0, :, 0] + c_ref[1, :, 0]
    invc = 1.0 / jnp.maximum(cnt, 1.0)
    m = (p_ref[0] + p_ref[1]) * invc[:, None]
    h = _dot_t(m, wl_ref[...]) + bl_ref[...][None, :] + _dot_t(x_ref[...], wr_ref[...])
    h_ref[...] = jnp.maximum(h, 0.0)
    ic_ref[...] = invc[:, None]


def _dense2_body(p_ref, ic_ref, h_ref, wl_ref, bl_ref, wr_ref, o_ref):
    m = (p_ref[0] + p_ref[1]) * ic_ref[...]
    o_ref[...] = _dot_t(m, wl_ref[...]) + bl_ref[...][None, :] + _dot_t(h_ref[...], wr_ref[...])


def _dense1(parts, cnts, x, Wl, bl, Wr):
    return pl.pallas_call(
        _dense1_body,
        grid=(N // BN,),
        in_specs=[
            pl.BlockSpec((NC, BN, D), lambda i: (0, i, 0)),
            pl.BlockSpec((NC, BN, D), lambda i: (0, i, 0)),
            pl.BlockSpec((BN, D), lambda i: (i, 0)),
            pl.BlockSpec((D, D), lambda i: (0, 0)),
            pl.BlockSpec((D,), lambda i: (0,)),
            pl.BlockSpec((D, D), lambda i: (0, 0)),
        ],
        out_specs=[
            pl.BlockSpec((BN, D), lambda i: (i, 0)),
            pl.BlockSpec((BN, 1), lambda i: (i, 0)),
        ],
        out_shape=[
            jax.ShapeDtypeStruct((N, D), jnp.float32),
            jax.ShapeDtypeStruct((N, 1), jnp.float32),
        ],
    )(parts, cnts, x, Wl, bl, Wr)


def _dense2(parts, invc, h, Wl, bl, Wr):
    return pl.pallas_call(
        _dense2_body,
        grid=(N // BN,),
        in_specs=[
            pl.BlockSpec((NC, BN, D), lambda i: (0, i, 0)),
            pl.BlockSpec((BN, 1), lambda i: (i, 0)),
            pl.BlockSpec((BN, D), lambda i: (i, 0)),
            pl.BlockSpec((D, D), lambda i: (0, 0)),
            pl.BlockSpec((D,), lambda i: (0,)),
            pl.BlockSpec((D, D), lambda i: (0, 0)),
        ],
        out_specs=pl.BlockSpec((BN, D), lambda i: (i, 0)),
        out_shape=jax.ShapeDtypeStruct((N, D), jnp.float32),
    )(parts, invc, h, Wl, bl, Wr)


def kernel(x, edge_index, W1l, b1l, W1r, W2l, b2l, W2r):
    src = edge_index[0].astype(jnp.int32)
    dst = edge_index[1].astype(jnp.int32)
    pad = EP - E
    pad_ar = jnp.arange(pad, dtype=jnp.int32)
    pad_src = (pad_ar * 97) % N
    pad_dst = N + pad_ar % (NP - N)
    src_flat = jnp.concatenate([src, pad_src])
    dst_flat = jnp.concatenate([dst, pad_dst])
    dst = dst_flat.reshape(NW, CPT, CHUNK)
    src2 = src_flat.reshape(NW, CPT2, CH2)
    dst2 = dst_flat.reshape(NW, CPT2, CH2)
    zeros_d = jnp.zeros((CHUNK, D), jnp.float32)
    zeros_d2 = jnp.zeros((CH2, D), jnp.float32)

    ones_d = jnp.ones((CHUNK, D), jnp.float32)
    cnts = _sc_ones(dst, ones_d, zeros_d)
    parts1 = _sc_agg(x, src2, dst2, zeros_d2)
    h, invc = _dense1(parts1, cnts, x, W1l, b1l, W1r)
    parts2 = _sc_agg(h, src2, dst2, zeros_d2)
    out = _dense2(parts2, invc, h, W2l, b2l, W2r)
    return out


# speedup vs baseline: 5.9171x; 1.0006x over previous
import jax
import jax.numpy as jnp
from jax import lax
from jax.experimental import pallas as pl
from jax.experimental.pallas import tpu as pltpu
from jax.experimental.pallas import tpu_sc as plsc

N = 10000
E = 320000
D = 128

NC = 2
NS = 16
NW = NC * NS

CHUNK = 128
CPT = 80
EP = NW * CPT * CHUNK
NP = 10112
NBLK = NP // CHUNK
KMAX = (NBLK + NS - 1) // NS

_MESH = plsc.VectorSubcoreMesh(core_axis_name="c", subcore_axis_name="s")


CH2 = 64
CPT2 = EP // (NW * CH2)
NB2 = NP // CH2
KMAX2 = (NB2 + NS - 1) // NS


def _sc_agg_body(table_hbm, src_hbm, dst_hbm, zeros_hbm, parts_hbm,
                 sidx0, didx0, rbuf0, gsem0, sidx1, didx1, rbuf1, gsem1, acc):
    cid = lax.axis_index("c")
    sid = lax.axis_index("s")
    wid = sid * NC + cid
    pltpu.sync_copy(zeros_hbm, rbuf0)
    pltpu.sync_copy(zeros_hbm, rbuf1)

    def zero_body(k, c):
        b = sid + k * NS

        @pl.when(b < NB2)
        def _():
            pltpu.sync_copy(rbuf0, acc.at[pl.ds(b * CH2, CH2)])

        return c

    lax.fori_loop(0, KMAX2, zero_body, 0)
    plsc.subcore_barrier()

    pltpu.sync_copy(src_hbm.at[wid, 0], sidx0)
    pltpu.sync_copy(dst_hbm.at[wid, 0], didx0)
    pltpu.async_copy(table_hbm.at[sidx0], rbuf0, gsem0)

    def pair_body(g, c):
        j1 = 2 * g + 1
        pltpu.sync_copy(src_hbm.at[wid, j1], sidx1)
        pltpu.sync_copy(dst_hbm.at[wid, j1], didx1)
        pltpu.async_copy(table_hbm.at[sidx1], rbuf1, gsem1)
        pltpu.make_async_copy(table_hbm.at[sidx0], rbuf0, gsem0).wait()
        pltpu.sync_copy(rbuf0, acc.at[didx0], add=True)

        @pl.when(j1 + 1 < CPT2)
        def _():
            pltpu.sync_copy(src_hbm.at[wid, j1 + 1], sidx0)
            pltpu.sync_copy(dst_hbm.at[wid, j1 + 1], didx0)
            pltpu.async_copy(table_hbm.at[sidx0], rbuf0, gsem0)

        pltpu.make_async_copy(table_hbm.at[sidx1], rbuf1, gsem1).wait()
        pltpu.sync_copy(rbuf1, acc.at[didx1], add=True)
        return c

    lax.fori_loop(0, CPT2 // 2, pair_body, 0)
    plsc.subcore_barrier()

    def out_body(k, c):
        b = sid + k * NS

        @pl.when(b < NB2)
        def _():
            r = b * CH2
            pltpu.sync_copy(acc.at[pl.ds(r, CH2)], rbuf0)
            pltpu.sync_copy(rbuf0, parts_hbm.at[cid, pl.ds(r, CH2)])

        return c

    lax.fori_loop(0, KMAX2, out_body, 0)


_sc_agg = pl.kernel(
    _sc_agg_body,
    out_type=jax.ShapeDtypeStruct((NC, NP, D), jnp.float32),
    mesh=_MESH,
    scratch_types=[
        pltpu.VMEM((CH2,), jnp.int32),
        pltpu.VMEM((CH2,), jnp.int32),
        pltpu.VMEM((CH2, D), jnp.float32),
        pltpu.SemaphoreType.DMA,
        pltpu.VMEM((CH2,), jnp.int32),
        pltpu.VMEM((CH2,), jnp.int32),
        pltpu.VMEM((CH2, D), jnp.float32),
        pltpu.SemaphoreType.DMA,
        pltpu.VMEM_SHARED((NP, D), jnp.float32),
    ],
)


def _sc_ones_body(dst_hbm, ones_hbm, zeros_hbm, cnts_hbm, didx, rbuf, acc):
    cid = lax.axis_index("c")
    sid = lax.axis_index("s")
    wid = sid * NC + cid
    pltpu.sync_copy(zeros_hbm, rbuf)

    def zero_body(k, c):
        b = sid + k * NS

        @pl.when(b < NBLK)
        def _():
            pltpu.sync_copy(rbuf, acc.at[pl.ds(b * CHUNK, CHUNK)])

        return c

    lax.fori_loop(0, KMAX, zero_body, 0)
    pltpu.sync_copy(ones_hbm, rbuf)
    plsc.subcore_barrier()

    def chunk_body(j, c):
        pltpu.sync_copy(dst_hbm.at[wid, j], didx)
        pltpu.sync_copy(rbuf, acc.at[didx], add=True)
        return c

    lax.fori_loop(0, CPT, chunk_body, 0)
    plsc.subcore_barrier()

    def out_body(k, c):
        b = sid + k * NS

        @pl.when(b < NBLK)
        def _():
            r = b * CHUNK
            pltpu.sync_copy(acc.at[pl.ds(r, CHUNK)], rbuf)
            pltpu.sync_copy(rbuf, cnts_hbm.at[cid, pl.ds(r, CHUNK)])

        return c

    lax.fori_loop(0, KMAX, out_body, 0)


_sc_ones = pl.kernel(
    _sc_ones_body,
    out_type=jax.ShapeDtypeStruct((NC, NP, D), jnp.float32),
    mesh=_MESH,
    scratch_types=[
        pltpu.VMEM((CHUNK,), jnp.int32),
        pltpu.VMEM((CHUNK, D), jnp.float32),
        pltpu.VMEM_SHARED((NP, D), jnp.float32),
    ],
)


BN = 400


def _dot_t(a, w):
    return lax.dot_general(a, w, (((1,), (1,)), ((), ())),
                           preferred_element_type=jnp.float32)


def _dense1_body(p_ref, c_ref, x_ref, wl_ref, bl_ref, wr_ref, h_ref, ic_ref):
    cnt = c_ref[0, :, 0] + c_ref[1, :, 0]
    invc = 1.0 / jnp.maximum(cnt, 1.0)
    m = (p_ref[0] + p_ref[1]) * invc[:, None]
    h = _dot_t(m, wl_ref[...]) + bl_ref[...][None, :] + _dot_t(x_ref[...], wr_ref[...])
    h_ref[...] = jnp.maximum(h, 0.0)
    ic_ref[...] = invc[:, None]


def _dense2_body(p_ref, ic_ref, h_ref, wl_ref, bl_ref, wr_ref, o_ref):
    m = (p_ref[0] + p_ref[1]) * ic_ref[...]
    o_ref[...] = _dot_t(m, wl_ref[...]) + bl_ref[...][None, :] + _dot_t(h_ref[...], wr_ref[...])


def _dense1(parts, cnts, x, Wl, bl, Wr):
    return pl.pallas_call(
        _dense1_body,
        grid=(N // BN,),
        in_specs=[
            pl.BlockSpec((NC, BN, D), lambda i: (0, i, 0)),
            pl.BlockSpec((NC, BN, D), lambda i: (0, i, 0)),
            pl.BlockSpec((BN, D), lambda i: (i, 0)),
            pl.BlockSpec((D, D), lambda i: (0, 0)),
            pl.BlockSpec((D,), lambda i: (0,)),
            pl.BlockSpec((D, D), lambda i: (0, 0)),
        ],
        out_specs=[
            pl.BlockSpec((BN, D), lambda i: (i, 0)),
            pl.BlockSpec((BN, 1), lambda i: (i, 0)),
        ],
        out_shape=[
            jax.ShapeDtypeStruct((N, D), jnp.float32),
            jax.ShapeDtypeStruct((N, 1), jnp.float32),
        ],
    )(parts, cnts, x, Wl, bl, Wr)


def _dense2(parts, invc, h, Wl, bl, Wr):
    return pl.pallas_call(
        _dense2_body,
        grid=(N // BN,),
        in_specs=[
            pl.BlockSpec((NC, BN, D), lambda i: (0, i, 0)),
            pl.BlockSpec((BN, 1), lambda i: (i, 0)),
            pl.BlockSpec((BN, D), lambda i: (i, 0)),
            pl.BlockSpec((D, D), lambda i: (0, 0)),
            pl.BlockSpec((D,), lambda i: (0,)),
            pl.BlockSpec((D, D), lambda i: (0, 0)),
        ],
        out_specs=pl.BlockSpec((BN, D), lambda i: (i, 0)),
        out_shape=jax.ShapeDtypeStruct((N, D), jnp.float32),
    )(parts, invc, h, Wl, bl, Wr)


def kernel(x, edge_index, W1l, b1l, W1r, W2l, b2l, W2r):
    src = edge_index[0].astype(jnp.int32)
    dst = edge_index[1].astype(jnp.int32)
    pad = EP - E
    pad_ar = jnp.arange(pad, dtype=jnp.int32)
    pad_src = (pad_ar * 97) % N
    pad_dst = N + pad_ar % (NP - N)
    src_flat = jnp.concatenate([src, pad_src])
    dst_flat = jnp.concatenate([dst, pad_dst])
    dst = dst_flat.reshape(NW, CPT, CHUNK)
    src2 = src_flat.reshape(NW, CPT2, CH2)
    dst2 = dst_flat.reshape(NW, CPT2, CH2)
    zeros_d = jnp.zeros((CHUNK, D), jnp.float32)
    zeros_d2 = jnp.zeros((CH2, D), jnp.float32)

    ones_d = jnp.ones((CHUNK, D), jnp.float32)
    cnts = _sc_ones(dst, ones_d, zeros_d)
    parts1 = _sc_agg(x, src2, dst2, zeros_d2)
    h, invc = _dense1(parts1, cnts, x, W1l, b1l, W1r)
    parts2 = _sc_agg(h, src2, dst2, zeros_d2)
    out = _dense2(parts2, invc, h, W2l, b2l, W2r)
    return out
